# baseline (speedup 1.0000x reference)
import functools

import jax
import jax.numpy as jnp
from jax import lax
from jax.experimental import pallas as pl
from jax.experimental.pallas import tpu as pltpu
from jax.experimental.pallas import tpu_sc as plsc

B = 4096
L = 50
V = 100000
D = 128
H = 128
C = 4

LP = 56
NW = 32
BPW = B // NW
CH = 2
NIDX = CH * LP
NCH = BPW // CH
NLG = D // 16


def _pool_body(xflat_hbm, table_hbm, out_hbm, idx_v, rows_v, out_v, sem):
    wid = lax.axis_index("s") * 2 + lax.axis_index("c")
    base = wid * BPW

    def chunk(c, carry):
        row0 = base + c * CH
        pltpu.sync_copy(xflat_hbm.at[pl.ds(row0 * LP, NIDX)], idx_v)
        pltpu.async_copy(table_hbm.at[idx_v], rows_v, sem).wait()
        for r in range(CH):
            o = r * LP

            def tok(j, acc):
                return tuple(
                    acc[k] + rows_v[o + j, pl.ds(16 * k, 16)] for k in range(NLG)
                )

            acc0 = tuple(rows_v[o, pl.ds(16 * k, 16)] for k in range(NLG))
            acc = lax.fori_loop(1, LP, tok, acc0, unroll=True)
            for k in range(NLG):
                out_v[r, pl.ds(16 * k, 16)] = acc[k]
        pltpu.sync_copy(out_v, out_hbm.at[pl.ds(row0, CH)])
        return carry

    lax.fori_loop(0, NCH, chunk, 0)


@functools.cache
def _pool():
    return functools.partial(
        pl.kernel,
        mesh=plsc.VectorSubcoreMesh(core_axis_name="c", subcore_axis_name="s"),
        out_type=jax.ShapeDtypeStruct((B, D), jnp.float32),
        scratch_types=[
            pltpu.VMEM((NIDX,), jnp.int32),
            pltpu.VMEM((NIDX, D), jnp.float32),
            pltpu.VMEM((CH, D), jnp.float32),
            pltpu.SemaphoreType.DMA,
        ],
    )(_pool_body)


def _mlp_body(s_ref, xl_ref, w1_ref, b1_ref, g_ref, bt_ref, mu_ref, var_ref,
              w2_ref, b2_ref, out_ref):
    s = s_ref[...]
    z = jnp.dot(s, w1_ref[...], preferred_element_type=jnp.float32)
    z = z / xl_ref[...] + b1_ref[...]
    a = g_ref[...] * lax.rsqrt(var_ref[...] + 1e-5)
    cshift = bt_ref[...] - mu_ref[...] * a
    h = jnp.maximum(z * a + cshift, 0.0)
    logits = jnp.dot(h, w2_ref[...], preferred_element_type=jnp.float32)
    logits = logits + b2_ref[...]
    col = lax.broadcasted_iota(jnp.int32, logits.shape, 1)
    valid = col < C
    masked = jnp.where(valid, logits, -jnp.inf)
    m = jnp.max(masked, axis=1, keepdims=True)
    e = jnp.where(valid, jnp.exp(logits - m), 0.0)
    lse = m + jnp.log(jnp.sum(e, axis=1, keepdims=True))
    out_ref[...] = logits - lse


def _mlp(s, xl, W1, b1, gamma, beta, mu, var, W2p, b2p):
    return pl.pallas_call(
        _mlp_body,
        out_shape=jax.ShapeDtypeStruct((B, D), jnp.float32),
    )(s, xl, W1, b1, gamma, beta, mu, var, W2p, b2p)


def kernel(x, x_len, table, W1, b1, gamma, beta, run_mean, run_var, W2, b2):
    xpad = jnp.zeros((B, LP), jnp.int32).at[:, :L].set(x.astype(jnp.int32))
    xflat = xpad.reshape(B * LP)
    s = _pool()(xflat, table)
    xl = x_len.astype(jnp.float32).reshape(B, 1)
    W2p = jnp.zeros((H, D), jnp.float32).at[:, :C].set(W2)
    b2p = jnp.zeros((1, D), jnp.float32).at[0, :C].set(b2)
    out = _mlp(s, xl, W1, b1.reshape(1, H), gamma.reshape(1, H),
               beta.reshape(1, H), run_mean.reshape(1, H),
               run_var.reshape(1, H), W2p, b2p)
    return out[:, :C]
